# baseline (speedup 1.0000x reference)
import jax
import jax.numpy as jnp
from jax.experimental import pallas as pl
from jax.experimental.pallas import tpu as pltpu

HEIGHT = 32
WIDTH = 32
CH = 192
B = 64
HW = HEIGHT * WIDTH

_NB = 2


def _body(x_ref, h_ref, w_ref, o_ref, pos_ref):
    @pl.when(pl.program_id(0) == 0)
    def _build_pos():
        row = jax.lax.broadcasted_iota(jnp.int32, (HEIGHT, HW), 0)
        p = jax.lax.broadcasted_iota(jnp.int32, (HEIGHT, HW), 1)
        eh = (p // WIDTH == row).astype(jnp.float32)
        ew = (p % WIDTH == row).astype(jnp.float32)
        dn = (((0,), (0,)), ((), ()))
        pos_ref[...] = jax.lax.dot_general(
            h_ref[...], eh, dn, preferred_element_type=jnp.float32
        ) + jax.lax.dot_general(
            w_ref[...], ew, dn, preferred_element_type=jnp.float32
        )

    o_ref[...] = x_ref[...] + pos_ref[...][None]


def kernel(x, h_emb, w_emb):
    b, c, h, w = x.shape
    xf = x.reshape(b, c, h * w)
    out = pl.pallas_call(
        _body,
        grid=(b // _NB,),
        in_specs=[
            pl.BlockSpec((_NB, c, h * w), lambda i: (i, 0, 0)),
            pl.BlockSpec((HEIGHT, CH), lambda i: (0, 0)),
            pl.BlockSpec((WIDTH, CH), lambda i: (0, 0)),
        ],
        out_specs=pl.BlockSpec((_NB, c, h * w), lambda i: (i, 0, 0)),
        out_shape=jax.ShapeDtypeStruct((b, c, h * w), jnp.float32),
        scratch_shapes=[pltpu.VMEM((CH, HW), jnp.float32)],
    )(xf, h_emb, w_emb)
    return out.reshape(b, c, h, w)


# speedup vs baseline: 1.0708x; 1.0708x over previous
import jax
import jax.numpy as jnp
from jax.experimental import pallas as pl
from jax.experimental.pallas import tpu as pltpu

HEIGHT = 32
WIDTH = 32
CH = 192
B = 64
HW = HEIGHT * WIDTH

_NB = 16


def _body(x_ref, h_ref, w_ref, o_ref, pos_ref):
    @pl.when(pl.program_id(0) == 0)
    def _build_pos():
        row = jax.lax.broadcasted_iota(jnp.int32, (HEIGHT, HW), 0)
        p = jax.lax.broadcasted_iota(jnp.int32, (HEIGHT, HW), 1)
        eh = (p // WIDTH == row).astype(jnp.float32)
        ew = (p % WIDTH == row).astype(jnp.float32)
        dn = (((0,), (0,)), ((), ()))
        pos_ref[...] = jax.lax.dot_general(
            h_ref[...], eh, dn, preferred_element_type=jnp.float32
        ) + jax.lax.dot_general(
            w_ref[...], ew, dn, preferred_element_type=jnp.float32
        )

    o_ref[...] = x_ref[...] + pos_ref[...][None]


def kernel(x, h_emb, w_emb):
    b, c, h, w = x.shape
    xf = x.reshape(b, c, h * w)
    out = pl.pallas_call(
        _body,
        grid=(b // _NB,),
        in_specs=[
            pl.BlockSpec((_NB, c, h * w), lambda i: (i, 0, 0)),
            pl.BlockSpec((HEIGHT, CH), lambda i: (0, 0)),
            pl.BlockSpec((WIDTH, CH), lambda i: (0, 0)),
        ],
        out_specs=pl.BlockSpec((_NB, c, h * w), lambda i: (i, 0, 0)),
        out_shape=jax.ShapeDtypeStruct((b, c, h * w), jnp.float32),
        scratch_shapes=[pltpu.VMEM((CH, HW), jnp.float32)],
    )(xf, h_emb, w_emb)
    return out.reshape(b, c, h, w)
